# baseline (speedup 1.0000x reference)
import functools

import jax
import jax.numpy as jnp
from jax import lax
from jax.experimental import pallas as pl
from jax.experimental.pallas import tpu as pltpu
from jax.experimental.pallas import tpu_sc as plsc

_MSP = 800
_T = 102
_D = 256
_PAD = 799
_J = 4
_B = 16384

_NC = 2
_NS = 16
_NW = _NC * _NS
_L = 16

_NP = _J // 2
_CHUNK = _B // (_NW // _NP)
_SUB = 128
_NSUB = _CHUNK // _SUB
_NG = _SUB // _L


_TP = 104


def _tc_table_body(emb_ref, wd_ref, b_ref, out_ref):
    row = lax.broadcasted_iota(jnp.int32, (_MSP, _D), 0)
    embz = jnp.where(row == _PAD, 0.0, emb_ref[...])
    p3 = lax.dot_general(
        wd_ref[...], embz, (((0,), (1,)), ((), ())),
        preferred_element_type=jnp.float32,
    ) + b_ref[...]
    for p in range(_NP):
        lo = p3[p * 2 * _TP : p * 2 * _TP + _TP]
        hi = p3[p * 2 * _TP + _TP : (p + 1) * 2 * _TP]
        lo16 = lax.bitcast_convert_type(lo.astype(jnp.bfloat16), jnp.uint16)
        hi16 = lax.bitcast_convert_type(hi.astype(jnp.bfloat16), jnp.uint16)
        packed = lo16.astype(jnp.uint32) | (hi16.astype(jnp.uint32) << 16)
        out_ref[p] = lax.bitcast_convert_type(packed, jnp.int32)


def _build_table(emb, W, b):
    wd3 = W.reshape(_J, _T, _D).transpose(2, 0, 1)
    wd3 = jnp.pad(wd3, ((0, 0), (0, 0), (0, _TP - _T)))
    perm = jnp.concatenate([jnp.arange(0, _TP, 2), jnp.arange(1, _TP, 2)])
    wd3 = wd3[:, :, perm].reshape(_D, _J * _TP)
    brow = jnp.zeros((_J * _TP,), jnp.float32).at[jnp.arange(_J) * _TP].set(b)
    packed = pl.pallas_call(
        _tc_table_body,
        out_shape=jax.ShapeDtypeStruct((_NP, _TP, _MSP), jnp.int32),
    )(emb, wd3, brow.reshape(_J * _TP, 1))
    return packed.reshape(_NP, _TP * _MSP)


_TPP = 51
_PB = 52 * _MSP
_P1 = 52 * _MSP
_TSPL = 28


def _sc_body(table_hbm, srcT_hbm, out_hbm, tab_v, src_v0, src_v1,
             a00, a01, a10, a11, sem0, sem1, tsem0, tsem1):
    par = lax.axis_index("c")
    b0 = lax.axis_index("s") * _CHUNK
    poff = par * _PB

    sems = (sem0, sem1)
    srcs = (src_v0, src_v1)
    himask = jnp.full((_L,), -65536, jnp.int32)

    def start(s):
        return pltpu.async_copy(
            srcT_hbm.at[:, pl.ds(b0 + s * _SUB, _SUB)], srcs[s % 2], sems[s % 2]
        )

    pending = start(0)
    trow0 = table_hbm.at[0]
    trow1 = table_hbm.at[1]
    n1 = _TSPL * _MSP
    n2 = (52 - _TSPL) * _MSP
    c1 = pltpu.async_copy(trow0.at[pl.ds(poff, n1)], tab_v.at[pl.ds(0, n1)], tsem0)
    c2 = pltpu.async_copy(trow1.at[pl.ds(poff, n1)], tab_v.at[pl.ds(_P1, n1)], tsem0)
    c3 = pltpu.async_copy(
        trow0.at[pl.ds(poff + n1, n2)], tab_v.at[pl.ds(n1, n2)], tsem1
    )
    c4 = pltpu.async_copy(
        trow1.at[pl.ds(poff + n1, n2)], tab_v.at[pl.ds(_P1 + n1, n2)], tsem1
    )

    c1.wait()
    c2.wait()
    for s in range(_NSUB):
        buf = s % 2
        nxt = start(s + 1) if s + 1 < _NSUB else None
        pending.wait()
        pending = nxt

        sbuf = srcs[buf]

        def tp_body(tp, accs):
            l0a, h0a, l1a, h1a = accs
            toff = tp * _MSP
            trow = 2 * tp + par
            l0o, h0o, l1o, h1o = [], [], [], []
            for g in range(_NG):
                iv = sbuf[trow, pl.ds(g * _L, _L)]
                i0 = iv + toff
                w0 = plsc.load_gather(tab_v, [i0])
                w1 = plsc.load_gather(tab_v, [i0 + _P1])
                l0o.append(l0a[g] + plsc.bitcast(lax.shift_left(w0, 16), jnp.float32))
                h0o.append(h0a[g] + plsc.bitcast(w0 & himask, jnp.float32))
                l1o.append(l1a[g] + plsc.bitcast(lax.shift_left(w1, 16), jnp.float32))
                h1o.append(h1a[g] + plsc.bitcast(w1 & himask, jnp.float32))
            return tuple(l0o), tuple(h0o), tuple(l1o), tuple(h1o)

        zeros = tuple(jnp.zeros((_L,), jnp.float32) for _ in range(_NG))
        accs = lax.fori_loop(0, _TSPL, tp_body, (zeros, zeros, zeros, zeros))
        if s == 0:
            c3.wait()
            c4.wait()
        l0a, h0a, l1a, h1a = lax.fori_loop(_TSPL, _TPP, tp_body, accs)
        for g in range(_NG):
            a00[pl.ds(s * _SUB + g * _L, _L)] = l0a[g]
            a01[pl.ds(s * _SUB + g * _L, _L)] = h0a[g]
            a10[pl.ds(s * _SUB + g * _L, _L)] = l1a[g]
            a11[pl.ds(s * _SUB + g * _L, _L)] = h1a[g]

    pltpu.sync_copy(a00, out_hbm.at[par, 0, pl.ds(b0, _CHUNK)])
    pltpu.sync_copy(a01, out_hbm.at[par, 1, pl.ds(b0, _CHUNK)])
    pltpu.sync_copy(a10, out_hbm.at[par, 2, pl.ds(b0, _CHUNK)])
    pltpu.sync_copy(a11, out_hbm.at[par, 3, pl.ds(b0, _CHUNK)])


_sc_lookup = functools.partial(
    pl.kernel,
    out_type=jax.ShapeDtypeStruct((2, _J, _B), jnp.float32),
    mesh=plsc.VectorSubcoreMesh(core_axis_name="c", subcore_axis_name="s"),
    compiler_params=pltpu.CompilerParams(needs_layout_passes=False),
    scratch_types=[
        pltpu.VMEM((2 * 52 * _MSP,), jnp.int32),
        pltpu.VMEM((_T, _SUB), jnp.int32),
        pltpu.VMEM((_T, _SUB), jnp.int32),
        pltpu.VMEM((_CHUNK,), jnp.float32),
        pltpu.VMEM((_CHUNK,), jnp.float32),
        pltpu.VMEM((_CHUNK,), jnp.float32),
        pltpu.VMEM((_CHUNK,), jnp.float32),
        pltpu.SemaphoreType.DMA,
        pltpu.SemaphoreType.DMA,
        pltpu.SemaphoreType.DMA,
        pltpu.SemaphoreType.DMA,
    ],
)(_sc_body)


def kernel(src, emb, W, b):
    table = _build_table(emb, W, b)
    srcT = src.T
    out2 = _sc_lookup(table, srcT)
    return (out2[0] + out2[1]).T


# speedup vs baseline: 1.0413x; 1.0413x over previous
import functools

import jax
import jax.numpy as jnp
from jax import lax
from jax.experimental import pallas as pl
from jax.experimental.pallas import tpu as pltpu
from jax.experimental.pallas import tpu_sc as plsc

_MSP = 800
_T = 102
_D = 256
_PAD = 799
_J = 4
_B = 16384

_NC = 2
_NS = 16
_NW = _NC * _NS
_L = 16

_NP = _J // 2
_CHUNK = _B // (_NW // _NP)
_SUB = 128
_NSUB = _CHUNK // _SUB
_NG = _SUB // _L


_TP = 104


def _tc_table_body(emb_ref, wd_ref, b_ref, out_ref):
    row = lax.broadcasted_iota(jnp.int32, (_MSP, _D), 0)
    embz = jnp.where(row == _PAD, 0.0, emb_ref[...])
    p3 = lax.dot_general(
        wd_ref[...], embz, (((0,), (1,)), ((), ())),
        preferred_element_type=jnp.float32,
    ) + b_ref[...]
    for p in range(_NP):
        lo = p3[p * 2 * _TP : p * 2 * _TP + _TP]
        hi = p3[p * 2 * _TP + _TP : (p + 1) * 2 * _TP]
        lo16 = lax.bitcast_convert_type(lo.astype(jnp.bfloat16), jnp.uint16)
        hi16 = lax.bitcast_convert_type(hi.astype(jnp.bfloat16), jnp.uint16)
        packed = lo16.astype(jnp.uint32) | (hi16.astype(jnp.uint32) << 16)
        out_ref[p] = lax.bitcast_convert_type(packed, jnp.int32)


def _build_table(emb, W, b):
    wd3 = W.reshape(_J, _T, _D).transpose(2, 0, 1)
    wd3 = jnp.pad(wd3, ((0, 0), (0, 0), (0, _TP - _T))).reshape(_D, _J * _TP)
    brow = jnp.zeros((_J * _TP,), jnp.float32).at[jnp.arange(_J) * _TP].set(b)
    packed = pl.pallas_call(
        _tc_table_body,
        out_shape=jax.ShapeDtypeStruct((_NP, _TP, _MSP), jnp.int32),
    )(emb, wd3, brow.reshape(_J * _TP, 1))
    return packed.reshape(_NP, _TP * _MSP)


_TSPLIT = 52
_H1 = _TSPLIT * _MSP
_H2 = _TP * _MSP - _H1


def _sc_body(table_hbm, srcT_hbm, out_hbm, tab_v, src_v0, src_v1, acc0_v, acc1_v,
             sem0, sem1, tsem0, tsem1):
    wid = lax.axis_index("s") * _NC + lax.axis_index("c")
    p = wid % _NP
    c = wid // _NP
    b0 = c * _CHUNK

    sems = (sem0, sem1)
    srcs = (src_v0, src_v1)
    himask = jnp.full((_L,), -65536, jnp.int32)

    def start(s):
        return pltpu.async_copy(
            srcT_hbm.at[:, pl.ds(b0 + s * _SUB, _SUB)], srcs[s % 2], sems[s % 2]
        )

    pending = start(0)
    trow = table_hbm.at[p]
    tcopy0 = pltpu.async_copy(
        trow.at[pl.ds(0, _H1)], tab_v.at[pl.ds(0, _H1)], tsem0
    )
    tcopy1 = pltpu.async_copy(
        trow.at[pl.ds(_H1, _H2)], tab_v.at[pl.ds(_H1, _H2)], tsem1
    )

    tcopy0.wait()
    for s in range(_NSUB):
        buf = s % 2
        nxt = start(s + 1) if s + 1 < _NSUB else None
        pending.wait()
        pending = nxt

        sbuf = srcs[buf]

        def t_body(t, accs):
            lo_accs, hi_accs = accs
            toff = t * _MSP
            lo_out, hi_out = [], []
            for g in range(_NG):
                iv = sbuf[t, pl.ds(g * _L, _L)]
                w = plsc.load_gather(tab_v, [iv + toff])
                lo = plsc.bitcast(lax.shift_left(w, 16), jnp.float32)
                hi = plsc.bitcast(w & himask, jnp.float32)
                lo_out.append(lo_accs[g] + lo)
                hi_out.append(hi_accs[g] + hi)
            return tuple(lo_out), tuple(hi_out)

        zeros = tuple(jnp.zeros((_L,), jnp.float32) for _ in range(_NG))
        accs = lax.fori_loop(0, _TSPLIT, t_body, (zeros, zeros), unroll=2)
        if s == 0:
            tcopy1.wait()
        lo_accs, hi_accs = lax.fori_loop(_TSPLIT, _T, t_body, accs, unroll=2)
        for g in range(_NG):
            acc0_v[pl.ds(s * _SUB + g * _L, _L)] = lo_accs[g]
            acc1_v[pl.ds(s * _SUB + g * _L, _L)] = hi_accs[g]

    pltpu.sync_copy(acc0_v, out_hbm.at[2 * p, pl.ds(b0, _CHUNK)])
    pltpu.sync_copy(acc1_v, out_hbm.at[2 * p + 1, pl.ds(b0, _CHUNK)])


_sc_lookup = functools.partial(
    pl.kernel,
    out_type=jax.ShapeDtypeStruct((_J, _B), jnp.float32),
    mesh=plsc.VectorSubcoreMesh(core_axis_name="c", subcore_axis_name="s"),
    compiler_params=pltpu.CompilerParams(needs_layout_passes=False),
    scratch_types=[
        pltpu.VMEM((_TP * _MSP,), jnp.int32),
        pltpu.VMEM((_T, _SUB), jnp.int32),
        pltpu.VMEM((_T, _SUB), jnp.int32),
        pltpu.VMEM((_CHUNK,), jnp.float32),
        pltpu.VMEM((_CHUNK,), jnp.float32),
        pltpu.SemaphoreType.DMA,
        pltpu.SemaphoreType.DMA,
        pltpu.SemaphoreType.DMA,
        pltpu.SemaphoreType.DMA,
    ],
)(_sc_body)


def kernel(src, emb, W, b):
    table = _build_table(emb, W, b)
    srcT = src.T
    outT = _sc_lookup(table, srcT)
    return outT.T
